# baseline (speedup 1.0000x reference)
import jax
import jax.numpy as jnp
from jax import lax
from jax.experimental import pallas as pl
from jax.experimental.pallas import tpu as pltpu
from jax.experimental.pallas import tpu_sc as plsc

M = 100000
D = 128
B = 16384
NC = 2
NS = 16
NW = NC * NS
RPW = B // NW
NB = 4
CH = RPW // NB
G = 8
NGC = CH // G
LW = 25
LS = M // LW
LSTEPS = LS // 16

_SC_PARAMS = pltpu.CompilerParams(use_tc_tiling_on_sc=False,
                                  needs_layout_passes=False)
_SC_MESH = plsc.VectorSubcoreMesh(core_axis_name="c", subcore_axis_name="s",
                                  num_cores=NC, num_subcores=NS)


def _wid():
    return lax.axis_index("c") * NS + lax.axis_index("s")


def _feat_body(feat_hbm, bf_hbm, ptr_hbm,
               fb0, fb1, fb2, fb3, ptr_v,
               is0, is1, is2, is3, os0, os1, os2, os3):
    fbufs = (fb0, fb1, fb2, fb3)
    in_sems = (is0, is1, is2, is3)
    out_sems = (os0, os1, os2, os3)

    base = _wid() * RPW

    for b in range(NB):
        pltpu.make_async_copy(bf_hbm.at[pl.ds(base + b * CH, CH)],
                              fbufs[b], in_sems[b]).start()

    pltpu.sync_copy(ptr_hbm, ptr_v)
    p = ptr_v[...][0]

    def dmod(x):
        t = p + x
        return jnp.where(t >= M, t - M, t)

    for b in range(NB):
        q = base + b * CH
        d = dmod(q)
        pltpu.make_async_copy(bf_hbm.at[pl.ds(0, CH)],
                              fbufs[b], in_sems[b]).wait()
        wraps = d > M - CH

        @pl.when(jnp.logical_not(wraps))
        def _():
            pltpu.make_async_copy(fbufs[b], feat_hbm.at[pl.ds(d, CH)],
                                  out_sems[b]).start()

        @pl.when(wraps)
        def _():
            def gbody(gi, carry):
                dg = dmod(q + gi * G)
                gwraps = dg > M - G

                @pl.when(jnp.logical_not(gwraps))
                def _():
                    pltpu.sync_copy(fbufs[b].at[pl.ds(gi * G, G)],
                                    feat_hbm.at[pl.ds(dg, G)])

                @pl.when(gwraps)
                def _():
                    for r in range(G):
                        dr = dmod(q + gi * G + r)
                        pltpu.sync_copy(fbufs[b].at[pl.ds(gi * G + r, 1)],
                                        feat_hbm.at[pl.ds(dr, 1)])

                return carry
            lax.fori_loop(0, NGC, gbody, 0)

    for b in range(NB):
        @pl.when(dmod(base + b * CH) <= M - CH)
        def _():
            pltpu.make_async_copy(fbufs[b], feat_hbm.at[pl.ds(0, CH)],
                                  out_sems[b]).wait()


_feat_scatter = pl.kernel(
    _feat_body,
    out_type=(),
    mesh=_SC_MESH,
    compiler_params=_SC_PARAMS,
    scratch_types=[
        pltpu.VMEM((CH, D), jnp.float32),
        pltpu.VMEM((CH, D), jnp.float32),
        pltpu.VMEM((CH, D), jnp.float32),
        pltpu.VMEM((CH, D), jnp.float32),
        pltpu.VMEM((16,), jnp.int32),
        pltpu.SemaphoreType.DMA,
        pltpu.SemaphoreType.DMA,
        pltpu.SemaphoreType.DMA,
        pltpu.SemaphoreType.DMA,
        pltpu.SemaphoreType.DMA,
        pltpu.SemaphoreType.DMA,
        pltpu.SemaphoreType.DMA,
        pltpu.SemaphoreType.DMA,
    ],
)


def _lab_body(ml_hbm, bl_hbm, ptr_hbm, outl_hbm, lab_v, bl_v, ptr_v,
              lsem0, lsem1):
    wid = _wid()
    s0 = wid * LS
    is_lab = wid < LW

    @pl.when(is_lab)
    def _():
        pltpu.make_async_copy(ml_hbm.at[pl.ds(s0, LS)], lab_v, lsem0).start()
        pltpu.make_async_copy(bl_hbm, bl_v, lsem1).start()

    pltpu.sync_copy(ptr_hbm, ptr_v)
    p = ptr_v[...][0]

    @pl.when(is_lab)
    def _():
        pltpu.make_async_copy(ml_hbm.at[pl.ds(0, LS)], lab_v, lsem0).wait()
        pltpu.make_async_copy(bl_hbm, bl_v, lsem1).wait()
        lanes = lax.iota(jnp.int32, 16)

        def lbody(i, carry):
            off = i * 16
            g = s0 + off + lanes
            t1 = g - p
            j = jnp.where(t1 < 0, t1 + M, t1)
            mask = j < B
            jc = jnp.where(mask, j, 0)
            gathered = plsc.load_gather(bl_v, [jc])
            cur = lab_v[pl.ds(off, 16)]
            lab_v[pl.ds(off, 16)] = jnp.where(mask, gathered, cur)
            return carry
        lax.fori_loop(0, LSTEPS, lbody, 0)
        pltpu.sync_copy(lab_v, outl_hbm.at[pl.ds(s0, LS)])


_lab_rewrite = pl.kernel(
    _lab_body,
    out_type=jax.ShapeDtypeStruct((M,), jnp.int32),
    mesh=_SC_MESH,
    compiler_params=_SC_PARAMS,
    scratch_types=[
        pltpu.VMEM((LS,), jnp.int32),
        pltpu.VMEM((B,), jnp.int32),
        pltpu.VMEM((16,), jnp.int32),
        pltpu.SemaphoreType.DMA,
        pltpu.SemaphoreType.DMA,
    ],
)


def kernel(memory_features, memory_labels, batch_features, batch_labels, ptr):
    ptr32 = jnp.asarray(ptr, jnp.int32)
    ptr_arr = jnp.full((16,), ptr32, dtype=jnp.int32)
    new_labels = _lab_rewrite(memory_labels, batch_labels, ptr_arr)
    feat_ref = jax.new_ref(memory_features)
    _feat_scatter(feat_ref, batch_features, ptr_arr)
    new_features = feat_ref[...]
    new_ptr = (ptr32 + B) % M
    return new_features, new_labels, new_ptr


# speedup vs baseline: 1.1402x; 1.1402x over previous
import jax
import jax.numpy as jnp
from jax import lax
from jax.experimental import pallas as pl
from jax.experimental.pallas import tpu as pltpu
from jax.experimental.pallas import tpu_sc as plsc

M = 100000
D = 128
B = 16384
NC = 2
NS = 16
NW = NC * NS
RPW = B // NW
NB = 4
CH = RPW // NB
G = 8
NGC = CH // G
LW = 25
LS = M // LW
LSTEPS = LS // 16


def _body(feat_hbm, ml_hbm, bf_hbm, bl_hbm, ptr_hbm, outl_hbm,
          fb0, fb1, fb2, fb3, lab_v, bl_v, ptr_v,
          is0, is1, is2, is3, os0, os1, os2, os3, lsem0, lsem1):
    fbufs = (fb0, fb1, fb2, fb3)
    in_sems = (is0, is1, is2, is3)
    out_sems = (os0, os1, os2, os3)

    cid = lax.axis_index("c")
    sid = lax.axis_index("s")
    wid = cid * NS + sid

    pltpu.sync_copy(ptr_hbm, ptr_v)
    p = ptr_v[...][0]

    base = wid * RPW
    s0 = wid * LS
    is_lab = wid < LW

    for b in range(NB):
        pltpu.make_async_copy(bf_hbm.at[pl.ds(base + b * CH, CH)],
                              fbufs[b], in_sems[b]).start()

    e = p + B - M
    w1lo = jnp.maximum(s0, p)
    w1hi = jnp.minimum(s0 + LS, jnp.minimum(p + B, M))
    ne1 = w1lo < w1hi
    w2hi = jnp.minimum(s0 + LS, e)
    ne2 = s0 < w2hi
    ne_any = jnp.logical_or(ne1, ne2)
    lo = jnp.where(ne2, s0, jnp.where(ne1, w1lo, s0 + LS))
    hi = jnp.maximum(jnp.where(ne1, w1hi, s0), jnp.where(ne2, w2hi, s0))
    step_lo = (lo - s0) // 16
    step_hi = jnp.maximum((hi - s0 + 15) // 16, step_lo)

    @pl.when(is_lab)
    def _():
        pltpu.make_async_copy(ml_hbm.at[pl.ds(s0, LS)], lab_v, lsem0).start()

        @pl.when(ne_any)
        def _():
            pltpu.make_async_copy(bl_hbm, bl_v, lsem1).start()

    def dmod(x):
        t = p + x
        return jnp.where(t >= M, t - M, t)

    for b in range(NB):
        q = base + b * CH
        d = dmod(q)
        pltpu.make_async_copy(bf_hbm.at[pl.ds(0, CH)],
                              fbufs[b], in_sems[b]).wait()
        wraps = d > M - CH

        @pl.when(jnp.logical_not(wraps))
        def _():
            pltpu.make_async_copy(fbufs[b], feat_hbm.at[pl.ds(d, CH)],
                                  out_sems[b]).start()

        @pl.when(wraps)
        def _():
            def gbody(gi, carry):
                dg = dmod(q + gi * G)
                gwraps = dg > M - G

                @pl.when(jnp.logical_not(gwraps))
                def _():
                    pltpu.sync_copy(fbufs[b].at[pl.ds(gi * G, G)],
                                    feat_hbm.at[pl.ds(dg, G)])

                @pl.when(gwraps)
                def _():
                    for r in range(G):
                        dr = dmod(q + gi * G + r)
                        pltpu.sync_copy(fbufs[b].at[pl.ds(gi * G + r, 1)],
                                        feat_hbm.at[pl.ds(dr, 1)])

                return carry
            lax.fori_loop(0, NGC, gbody, 0)

    @pl.when(is_lab)
    def _():
        pltpu.make_async_copy(ml_hbm.at[pl.ds(0, LS)], lab_v, lsem0).wait()

        @pl.when(ne_any)
        def _():
            pltpu.make_async_copy(bl_hbm, bl_v, lsem1).wait()
            lanes = lax.iota(jnp.int32, 16)

            def lbody(i, carry):
                off = i * 16
                g = s0 + off + lanes
                t1 = g - p
                j = jnp.where(t1 < 0, t1 + M, t1)
                mask = j < B
                jc = jnp.where(mask, j, 0)
                gathered = plsc.load_gather(bl_v, [jc])
                cur = lab_v[pl.ds(off, 16)]
                lab_v[pl.ds(off, 16)] = jnp.where(mask, gathered, cur)
                return carry
            lax.fori_loop(step_lo, step_hi, lbody, 0)
        pltpu.sync_copy(lab_v, outl_hbm.at[pl.ds(s0, LS)])

    for b in range(NB):
        @pl.when(dmod(base + b * CH) <= M - CH)
        def _():
            pltpu.make_async_copy(fbufs[b], feat_hbm.at[pl.ds(0, CH)],
                                  out_sems[b]).wait()


_scatter = pl.kernel(
    _body,
    out_type=jax.ShapeDtypeStruct((M,), jnp.int32),
    mesh=plsc.VectorSubcoreMesh(core_axis_name="c", subcore_axis_name="s",
                                num_cores=NC, num_subcores=NS),
    compiler_params=pltpu.CompilerParams(use_tc_tiling_on_sc=False,
                                         needs_layout_passes=False),
    scratch_types=[
        pltpu.VMEM((CH, D), jnp.float32),
        pltpu.VMEM((CH, D), jnp.float32),
        pltpu.VMEM((CH, D), jnp.float32),
        pltpu.VMEM((CH, D), jnp.float32),
        pltpu.VMEM((LS,), jnp.int32),
        pltpu.VMEM((B,), jnp.int32),
        pltpu.VMEM((16,), jnp.int32),
        pltpu.SemaphoreType.DMA,
        pltpu.SemaphoreType.DMA,
        pltpu.SemaphoreType.DMA,
        pltpu.SemaphoreType.DMA,
        pltpu.SemaphoreType.DMA,
        pltpu.SemaphoreType.DMA,
        pltpu.SemaphoreType.DMA,
        pltpu.SemaphoreType.DMA,
        pltpu.SemaphoreType.DMA,
        pltpu.SemaphoreType.DMA,
    ],
)


def kernel(memory_features, memory_labels, batch_features, batch_labels, ptr):
    ptr32 = jnp.asarray(ptr, jnp.int32)
    ptr_arr = jnp.full((16,), ptr32, dtype=jnp.int32)
    feat_ref = jax.new_ref(memory_features)
    new_labels = _scatter(feat_ref, memory_labels, batch_features,
                          batch_labels, ptr_arr)
    new_features = feat_ref[...]
    new_ptr = (ptr32 + B) % M
    return new_features, new_labels, new_ptr
